# baseline (speedup 1.0000x reference)
import functools

import jax
import jax.numpy as jnp
from jax import lax
from jax.experimental import pallas as pl
from jax.experimental.pallas import tpu as pltpu
from jax.experimental.pallas import tpu_sc as plsc

_B, _C = 16, 384
_T = 24 * 24
_K, _D = 1024, 32
_N = _B * _T
_EPS = 1e-5

_BB = 2
_T2 = _BB * _T
_NSTEP = _B // _BB

_NSC = _N // 2
_HSTEP = _NSTEP // 2

_NC, _NS = 2, 16
_NW = _NC * _NS
_BPW = _NSC // _NW


def _table_body(emb_ref, w1_ref, b1_ref, w2_ref, b2_ref, g_ref,
                bt_ref, out_ref, outbf_ref, embt_ref, e2_ref):
    bf = lambda a: a.astype(jnp.bfloat16)
    emb = emb_ref[...]
    h = jnp.maximum(
        jnp.dot(bf(emb), bf(w1_ref[...]),
                preferred_element_type=jnp.float32) + b1_ref[...], 0.0)
    q = jnp.dot(bf(h), bf(w2_ref[...]),
                preferred_element_type=jnp.float32) + b2_ref[...]
    m = jnp.mean(q, axis=-1, keepdims=True)
    v = jnp.mean((q - m) ** 2, axis=-1, keepdims=True)
    tab = (q - m) / jnp.sqrt(v + _EPS) * g_ref[...] + bt_ref[...]
    out_ref[...] = tab
    outbf_ref[...] = bf(tab)
    embt = emb.T
    embt_ref[...] = bf(embt)
    e2_ref[...] = jnp.sum(embt * embt, axis=0, keepdims=True)


def _table(emb, w1, b1, w2, b2, g, bt):
    return pl.pallas_call(
        _table_body,
        out_shape=[jax.ShapeDtypeStruct((_K, _C), jnp.float32),
                   jax.ShapeDtypeStruct((_K, _C), jnp.bfloat16),
                   jax.ShapeDtypeStruct((_D, _K), jnp.bfloat16),
                   jax.ShapeDtypeStruct((1, _K), jnp.float32)],
    )(emb, w1, b1, w2, b2, g, bt)


def _front_body(f_ref, w1_ref, b1_ref, w2_ref, b2_ref, g_ref, bt_ref,
                embt_ref, e2_ref, idx_ref):
    bf = lambda a: a.astype(jnp.bfloat16)
    w1 = w1_ref[...]
    ht = jnp.concatenate(
        [lax.dot_general(f_ref[i], w1, (((0,), (0,)), ((), ())),
                         preferred_element_type=jnp.float32)
         for i in range(_BB)], axis=0)
    h = jnp.maximum(ht + b1_ref[...], 0.0)
    z = jnp.dot(bf(h), w2_ref[...],
                preferred_element_type=jnp.float32) + b2_ref[...]
    m = jnp.mean(z, axis=-1, keepdims=True)
    v = jnp.mean((z - m) ** 2, axis=-1, keepdims=True)
    zn = (z - m) / jnp.sqrt(v + _EPS) * g_ref[...] + bt_ref[...]
    scores = e2_ref[...] - 2.0 * jnp.dot(bf(zn), embt_ref[...],
                                         preferred_element_type=jnp.float32)
    iota = lax.broadcasted_iota(jnp.int32, (_T2, _K), 1)
    mn = jnp.min(scores, axis=1, keepdims=True)
    idx = jnp.min(jnp.where(scores == mn, iota, _K), axis=1)
    idx_ref[0, 0, :] = idx


def _front(f3, w1, b1, w2, b2, g, bt, embt, e2):
    full = lambda *s: pl.BlockSpec(s, lambda i: (0,) * len(s))
    return pl.pallas_call(
        _front_body,
        grid=(_NSTEP,),
        in_specs=[
            pl.BlockSpec((_BB, _C, _T), lambda i: (i, 0, 0)),
            full(_C, _C), full(1, _C), full(_C, _D), full(1, _D),
            full(1, _D), full(1, _D), full(_D, _K), full(1, _K),
        ],
        out_specs=pl.BlockSpec((1, 1, _T2), lambda i: (i, 0, 0)),
        out_shape=jax.ShapeDtypeStruct((_NSTEP, 1, _T2), jnp.int32),
    )(f3, w1, b1, w2, b2, g, bt, embt, e2)


def _onehot_body(idx_ref, enc_ref):
    iota = lax.broadcasted_iota(jnp.int32, (_T2, _K), 1)
    enc_ref[0] = (iota == idx_ref[0, 0, :][:, None]).astype(jnp.float32)


def _onehot(idx3):
    return pl.pallas_call(
        _onehot_body,
        grid=(_NSTEP,),
        in_specs=[pl.BlockSpec((1, 1, _T2), lambda i: (i, 0, 0))],
        out_specs=pl.BlockSpec((1, _T2, _K), lambda i: (i, 0, 0)),
        out_shape=jax.ShapeDtypeStruct((_NSTEP, _T2, _K), jnp.float32),
    )(idx3)


def _qhi_body(idx_ref, tab_ref, q_ref):
    iota = lax.broadcasted_iota(jnp.int32, (_T2, _K), 1)
    oh = (iota == idx_ref[0, 0, :][:, None]).astype(
        jnp.float32).astype(jnp.bfloat16)
    q_ref[0] = jnp.dot(oh, tab_ref[...], preferred_element_type=jnp.float32)


def _qhi(idx3, table_bf):
    return pl.pallas_call(
        _qhi_body,
        grid=(_HSTEP,),
        in_specs=[pl.BlockSpec((1, 1, _T2), lambda i: (_HSTEP + i, 0, 0)),
                  pl.BlockSpec((_K, _C), lambda i: (0, 0))],
        out_specs=pl.BlockSpec((1, _T2, _C), lambda i: (i, 0, 0)),
        out_shape=jax.ShapeDtypeStruct((_HSTEP, _T2, _C), jnp.float32),
    )(idx3, table_bf)


_GC = 6
_GR = _BPW // _GC


def _gather_body(table_hbm, idx_hbm, out_hbm, idx_v, *bufs_sems):
    bufs = bufs_sems[:_GC]
    gsems = bufs_sems[_GC:2 * _GC]
    wsems = bufs_sems[2 * _GC:]
    wid = lax.axis_index("s") * _NC + lax.axis_index("c")
    base = wid * _BPW
    pltpu.sync_copy(idx_hbm.at[pl.ds(base, _BPW)], idx_v)
    gets = [pltpu.async_copy(table_hbm.at[idx_v.at[pl.ds(c * _GR, _GR)]],
                             bufs[c], gsems[c]) for c in range(_GC)]
    puts = []
    for c in range(_GC):
        gets[c].wait()
        puts.append(pltpu.async_copy(
            bufs[c], out_hbm.at[pl.ds(base + c * _GR, _GR)], wsems[c]))
    for p in puts:
        p.wait()


def _gather(table, idx):
    mesh = plsc.VectorSubcoreMesh(core_axis_name="c", subcore_axis_name="s")
    kern = functools.partial(
        pl.kernel, mesh=mesh,
        out_type=jax.ShapeDtypeStruct((_NSC, _C), jnp.float32),
        scratch_types=(
            [pltpu.VMEM((_BPW,), jnp.int32)]
            + [pltpu.VMEM((_GR, _C), jnp.float32) for _ in range(_GC)]
            + [pltpu.SemaphoreType.DMA for _ in range(2 * _GC)]
        ),
    )(_gather_body)
    return kern(table, idx)


def kernel(features, y, W_in1, b_in1, W_in2, b_in2, g_in, beta_in, emb,
           W_out1, b_out1, W_out2, b_out2, g_out, beta_out):
    row = lambda a: a.reshape(1, -1)
    bf = lambda a: a.astype(jnp.bfloat16)
    table, table_bf, embt_bf, e2 = _table(
        emb, W_out1, row(b_out1), W_out2, row(b_out2),
        row(g_out), row(beta_out))
    f3 = bf(features).reshape(_B, _C, _T)
    idx3 = _front(f3, bf(W_in1), row(b_in1), bf(W_in2), row(b_in2),
                  row(g_in), row(beta_in), embt_bf, e2)
    idx = idx3.reshape(_N)
    q_sc = _gather(table, idx[:_NSC])
    q_hi = _qhi(idx3, table_bf)
    enc = _onehot(idx3)
    quantized = jnp.concatenate(
        [q_sc.reshape(_B // 2, _T, _C), q_hi.reshape(_B // 2, _T, _C)], axis=0)
    return (quantized, idx.reshape(_N, 1), enc.reshape(_B, _T, _K))


# speedup vs baseline: 1.0314x; 1.0314x over previous
import functools

import jax
import jax.numpy as jnp
from jax import lax
from jax.experimental import pallas as pl
from jax.experimental.pallas import tpu as pltpu
from jax.experimental.pallas import tpu_sc as plsc

_B, _C = 16, 384
_T = 24 * 24
_K, _D = 1024, 32
_N = _B * _T
_EPS = 1e-5

_BB = 2
_T2 = _BB * _T
_NSTEP = _B // _BB

_NSC = _N // 2
_HSTEP = _NSTEP // 2

_NC, _NS = 2, 16
_NW = _NC * _NS
_BPW = _NSC // _NW


def _table_body(emb_ref, w1_ref, b1_ref, w2_ref, b2_ref, g_ref,
                bt_ref, out_ref, outbf_ref, embt_ref, e2_ref):
    bf = lambda a: a.astype(jnp.bfloat16)
    emb = emb_ref[...]
    h = jnp.maximum(
        jnp.dot(bf(emb), bf(w1_ref[...]),
                preferred_element_type=jnp.float32) + b1_ref[...], 0.0)
    q = jnp.dot(bf(h), bf(w2_ref[...]),
                preferred_element_type=jnp.float32) + b2_ref[...]
    m = jnp.mean(q, axis=-1, keepdims=True)
    v = jnp.mean((q - m) ** 2, axis=-1, keepdims=True)
    tab = (q - m) / jnp.sqrt(v + _EPS) * g_ref[...] + bt_ref[...]
    out_ref[...] = tab
    outbf_ref[...] = bf(tab)
    embt = emb.T
    embt_ref[...] = bf(embt)
    e2_ref[...] = jnp.sum(embt * embt, axis=0, keepdims=True)


def _table(emb, w1, b1, w2, b2, g, bt):
    return pl.pallas_call(
        _table_body,
        out_shape=[jax.ShapeDtypeStruct((_K, _C), jnp.float32),
                   jax.ShapeDtypeStruct((_K, _C), jnp.bfloat16),
                   jax.ShapeDtypeStruct((_D, _K), jnp.bfloat16),
                   jax.ShapeDtypeStruct((1, _K), jnp.float32)],
    )(emb, w1, b1, w2, b2, g, bt)


def _front_body(f_ref, w1_ref, b1_ref, w2_ref, b2_ref, g_ref, bt_ref,
                embt_ref, e2_ref, idx_ref):
    bf = lambda a: a.astype(jnp.bfloat16)
    w1 = w1_ref[...]
    ht = jnp.concatenate(
        [lax.dot_general(bf(f_ref[i]), w1, (((0,), (0,)), ((), ())),
                         preferred_element_type=jnp.float32)
         for i in range(_BB)], axis=0)
    h = jnp.maximum(ht + b1_ref[...], 0.0)
    z = jnp.dot(bf(h), w2_ref[...],
                preferred_element_type=jnp.float32) + b2_ref[...]
    m = jnp.mean(z, axis=-1, keepdims=True)
    v = jnp.mean((z - m) ** 2, axis=-1, keepdims=True)
    zn = (z - m) / jnp.sqrt(v + _EPS) * g_ref[...] + bt_ref[...]
    scores = e2_ref[...] - 2.0 * jnp.dot(bf(zn), embt_ref[...],
                                         preferred_element_type=jnp.float32)
    iota = lax.broadcasted_iota(jnp.int32, (_T2, _K), 1)
    mn = jnp.min(scores, axis=1, keepdims=True)
    idx = jnp.min(jnp.where(scores == mn, iota, _K), axis=1)
    idx_ref[0, 0, :] = idx


def _front(f3, w1, b1, w2, b2, g, bt, embt, e2):
    full = lambda *s: pl.BlockSpec(s, lambda i: (0,) * len(s))
    return pl.pallas_call(
        _front_body,
        grid=(_NSTEP,),
        in_specs=[
            pl.BlockSpec((_BB, _C, _T), lambda i: (i, 0, 0)),
            full(_C, _C), full(1, _C), full(_C, _D), full(1, _D),
            full(1, _D), full(1, _D), full(_D, _K), full(1, _K),
        ],
        out_specs=pl.BlockSpec((1, 1, _T2), lambda i: (i, 0, 0)),
        out_shape=jax.ShapeDtypeStruct((_NSTEP, 1, _T2), jnp.int32),
    )(f3, w1, b1, w2, b2, g, bt, embt, e2)


def _onehot_body(idx_ref, enc_ref):
    iota = lax.broadcasted_iota(jnp.int32, (_T2, _K), 1)
    enc_ref[0] = (iota == idx_ref[0, 0, :][:, None]).astype(jnp.float32)


def _onehot(idx3):
    return pl.pallas_call(
        _onehot_body,
        grid=(_NSTEP,),
        in_specs=[pl.BlockSpec((1, 1, _T2), lambda i: (i, 0, 0))],
        out_specs=pl.BlockSpec((1, _T2, _K), lambda i: (i, 0, 0)),
        out_shape=jax.ShapeDtypeStruct((_NSTEP, _T2, _K), jnp.float32),
    )(idx3)


def _qhi_body(idx_ref, tab_ref, q_ref):
    iota = lax.broadcasted_iota(jnp.int32, (_T2, _K), 1)
    oh = (iota == idx_ref[0, 0, :][:, None]).astype(
        jnp.float32).astype(jnp.bfloat16)
    q_ref[0] = jnp.dot(oh, tab_ref[...], preferred_element_type=jnp.float32)


def _qhi(idx3, table_bf):
    return pl.pallas_call(
        _qhi_body,
        grid=(_HSTEP,),
        in_specs=[pl.BlockSpec((1, 1, _T2), lambda i: (_HSTEP + i, 0, 0)),
                  pl.BlockSpec((_K, _C), lambda i: (0, 0))],
        out_specs=pl.BlockSpec((1, _T2, _C), lambda i: (i, 0, 0)),
        out_shape=jax.ShapeDtypeStruct((_HSTEP, _T2, _C), jnp.float32),
    )(idx3, table_bf)


def _gather_body(table_hbm, idx_hbm, out_hbm, idx_v, rows_v, sem):
    wid = lax.axis_index("s") * _NC + lax.axis_index("c")
    base = wid * _BPW
    pltpu.sync_copy(idx_hbm.at[pl.ds(base, _BPW)], idx_v)
    pltpu.async_copy(table_hbm.at[idx_v], rows_v, sem).wait()
    pltpu.sync_copy(rows_v, out_hbm.at[pl.ds(base, _BPW)])


def _gather(table, idx):
    mesh = plsc.VectorSubcoreMesh(core_axis_name="c", subcore_axis_name="s")
    kern = functools.partial(
        pl.kernel, mesh=mesh,
        out_type=jax.ShapeDtypeStruct((_NSC, _C), jnp.float32),
        scratch_types=[
            pltpu.VMEM((_BPW,), jnp.int32),
            pltpu.VMEM((_BPW, _C), jnp.float32),
            pltpu.SemaphoreType.DMA,
        ],
    )(_gather_body)
    return kern(table, idx)


def kernel(features, y, W_in1, b_in1, W_in2, b_in2, g_in, beta_in, emb,
           W_out1, b_out1, W_out2, b_out2, g_out, beta_out):
    row = lambda a: a.reshape(1, -1)
    bf = lambda a: a.astype(jnp.bfloat16)
    table, table_bf, embt_bf, e2 = _table(
        emb, W_out1, row(b_out1), W_out2, row(b_out2),
        row(g_out), row(beta_out))
    f3 = features.reshape(_B, _C, _T)
    idx3 = _front(f3, bf(W_in1), row(b_in1), bf(W_in2), row(b_in2),
                  row(g_in), row(beta_in), embt_bf, e2)
    idx = idx3.reshape(_N)
    q_sc = _gather(table, idx[:_NSC])
    q_hi = _qhi(idx3, table_bf)
    enc = _onehot(idx3)
    quantized = jnp.concatenate(
        [q_sc.reshape(_B // 2, _T, _C), q_hi.reshape(_B // 2, _T, _C)], axis=0)
    return (quantized, idx.reshape(_N, 1), enc.reshape(_B, _T, _K))


# speedup vs baseline: 1.0524x; 1.0204x over previous
import functools

import jax
import jax.numpy as jnp
from jax import lax
from jax.experimental import pallas as pl
from jax.experimental.pallas import tpu as pltpu
from jax.experimental.pallas import tpu_sc as plsc

_B, _C = 16, 384
_T = 24 * 24
_K, _D = 1024, 32
_N = _B * _T
_EPS = 1e-5

_BB = 2
_T2 = _BB * _T
_NSTEP = _B // _BB

_NC, _NS = 2, 16
_NW = _NC * _NS
_BPW = _N // _NW


def _table_body(emb_ref, w1_ref, b1_ref, w2_ref, b2_ref, g_ref,
                bt_ref, out_ref, embt_ref, e2_ref):
    bf = lambda a: a.astype(jnp.bfloat16)
    emb = emb_ref[...]
    h = jnp.maximum(
        jnp.dot(bf(emb), bf(w1_ref[...]),
                preferred_element_type=jnp.float32) + b1_ref[...], 0.0)
    q = jnp.dot(bf(h), bf(w2_ref[...]),
                preferred_element_type=jnp.float32) + b2_ref[...]
    m = jnp.mean(q, axis=-1, keepdims=True)
    v = jnp.mean((q - m) ** 2, axis=-1, keepdims=True)
    out_ref[...] = (q - m) / jnp.sqrt(v + _EPS) * g_ref[...] + bt_ref[...]
    embt = emb.T
    embt_ref[...] = bf(embt)
    e2_ref[...] = jnp.sum(embt * embt, axis=0, keepdims=True)


def _table(emb, w1, b1, w2, b2, g, bt):
    return pl.pallas_call(
        _table_body,
        out_shape=[jax.ShapeDtypeStruct((_K, _C), jnp.float32),
                   jax.ShapeDtypeStruct((_D, _K), jnp.bfloat16),
                   jax.ShapeDtypeStruct((1, _K), jnp.float32)],
    )(emb, w1, b1, w2, b2, g, bt)


def _front_body(f_ref, w1_ref, b1_ref, w2_ref, b2_ref, g_ref, bt_ref,
                embt_ref, e2_ref, idx_ref, enc_ref):
    bf = lambda a: a.astype(jnp.bfloat16)
    w1 = w1_ref[...]
    ht = jnp.concatenate(
        [lax.dot_general(bf(f_ref[i]), w1, (((0,), (0,)), ((), ())),
                         preferred_element_type=jnp.float32)
         for i in range(_BB)], axis=0)
    h = jnp.maximum(ht + b1_ref[...], 0.0)
    z = jnp.dot(bf(h), w2_ref[...],
                preferred_element_type=jnp.float32) + b2_ref[...]
    m = jnp.mean(z, axis=-1, keepdims=True)
    v = jnp.mean((z - m) ** 2, axis=-1, keepdims=True)
    zn = (z - m) / jnp.sqrt(v + _EPS) * g_ref[...] + bt_ref[...]
    scores = e2_ref[...] - 2.0 * jnp.dot(bf(zn), embt_ref[...],
                                         preferred_element_type=jnp.float32)
    iota = lax.broadcasted_iota(jnp.int32, (_T2, _K), 1)
    mn = jnp.min(scores, axis=1, keepdims=True)
    idx = jnp.min(jnp.where(scores == mn, iota, _K), axis=1)
    idx_ref[0, 0, :] = idx
    enc_ref[0] = (iota == idx[:, None]).astype(jnp.float32)


def _front(f3, w1, b1, w2, b2, g, bt, embt, e2):
    full = lambda *s: pl.BlockSpec(s, lambda i: (0,) * len(s))
    return pl.pallas_call(
        _front_body,
        grid=(_NSTEP,),
        in_specs=[
            pl.BlockSpec((_BB, _C, _T), lambda i: (i, 0, 0)),
            full(_C, _C), full(1, _C), full(_C, _D), full(1, _D),
            full(1, _D), full(1, _D), full(_D, _K), full(1, _K),
        ],
        out_specs=[
            pl.BlockSpec((1, 1, _T2), lambda i: (i, 0, 0)),
            pl.BlockSpec((1, _T2, _K), lambda i: (i, 0, 0)),
        ],
        out_shape=[
            jax.ShapeDtypeStruct((_NSTEP, 1, _T2), jnp.int32),
            jax.ShapeDtypeStruct((_NSTEP, _T2, _K), jnp.float32),
        ],
    )(f3, w1, b1, w2, b2, g, bt, embt, e2)


def _gather_body(table_hbm, idx_hbm, out_hbm, idx_v, rows_v, sem):
    wid = lax.axis_index("s") * _NC + lax.axis_index("c")
    base = wid * _BPW
    pltpu.sync_copy(idx_hbm.at[pl.ds(base, _BPW)], idx_v)
    pltpu.async_copy(table_hbm.at[idx_v], rows_v, sem).wait()
    pltpu.sync_copy(rows_v, out_hbm.at[pl.ds(base, _BPW)])


def _gather(table, idx):
    mesh = plsc.VectorSubcoreMesh(core_axis_name="c", subcore_axis_name="s")
    kern = functools.partial(
        pl.kernel, mesh=mesh,
        out_type=jax.ShapeDtypeStruct((_N, _C), jnp.float32),
        scratch_types=[
            pltpu.VMEM((_BPW,), jnp.int32),
            pltpu.VMEM((_BPW, _C), jnp.float32),
            pltpu.SemaphoreType.DMA,
        ],
    )(_gather_body)
    return kern(table, idx)


def kernel(features, y, W_in1, b_in1, W_in2, b_in2, g_in, beta_in, emb,
           W_out1, b_out1, W_out2, b_out2, g_out, beta_out):
    row = lambda a: a.reshape(1, -1)
    bf = lambda a: a.astype(jnp.bfloat16)
    table, embt_bf, e2 = _table(emb, W_out1, row(b_out1), W_out2, row(b_out2),
                                row(g_out), row(beta_out))
    f3 = features.reshape(_B, _C, _T)
    idx3, enc = _front(f3, bf(W_in1), row(b_in1), bf(W_in2), row(b_in2),
                       row(g_in), row(beta_in), embt_bf, e2)
    idx = idx3.reshape(_N)
    q = _gather(table, idx)
    quantized = q.reshape(_B, _T, _C)
    return (quantized, idx.reshape(_N, 1), enc.reshape(_B, _T, _K))
